# baseline (speedup 1.0000x reference)
import jax
import jax.numpy as jnp
from jax import lax
from jax.experimental import pallas as pl
from jax.experimental.pallas import tpu as pltpu
from jax.experimental.pallas import tpu_sc as plsc

_NC, _NS, _L = 2, 16, 16
_NW = _NC * _NS
_K = 8
_C = 64
_CH = 64
_NG = _K * _CH // 128


def _body(frag_hbm, alpha_hbm, table_hbm, out_hbm,
          idx0, idx1, alpha0, alpha1, rows0, rows1, out0, out1,
          gsem0, gsem1, osem0, osem1):
    tot = out_hbm.shape[0]
    wcb = frag_hbm.shape[3] // _CH
    nblk = frag_hbm.shape[2] * wcb
    cpw = tot // _NW
    wid = lax.axis_index("s") * _NC + lax.axis_index("c")
    base_cid = wid * cpw
    bufs = [(idx0, alpha0, rows0, out0, gsem0, osem0),
            (idx1, alpha1, rows1, out1, gsem1, osem1)]

    def load_and_fire(cid, buf):
        idx_v, alpha_v, rows_v = buf[0], buf[1], buf[2]
        n = cid // nblk
        blk = cid % nblk
        hh = blk // wcb
        w0 = (blk % wcb) * _CH
        pltpu.sync_copy(frag_hbm.at[n, :, hh, pl.ds(w0, _CH)], idx_v)
        pltpu.sync_copy(alpha_hbm.at[n, :, hh, pl.ds(w0, _CH)], alpha_v)
        for k in range(_K):
            pltpu.async_copy(table_hbm.at[idx_v.at[k]],
                             rows_v.at[pl.ds(k * _CH, _CH)], buf[4])

    def wait_gathers(buf):
        idx_v, rows_v = buf[0], buf[2]
        for k in range(_K):
            pltpu.make_async_copy(table_hbm.at[idx_v.at[k]],
                                  rows_v.at[pl.ds(k * _CH, _CH)],
                                  buf[4]).wait()

    def compute(buf):
        alpha_v, rows_v, out_v = buf[1], buf[2], buf[3]

        @plsc.parallel_loop(0, _CH // _L, unroll=2)
        def g_body(g):
            t = jnp.ones((_L,), jnp.float32)
            wk = []
            for k in range(_K):
                a = alpha_v[k, pl.ds(g * _L, _L)]
                wk.append(a * t)
                t = t * (1.0 - a)
            base = g * _L
            for j in range(_L):
                p = base + j
                ws = [wk[k][j] for k in range(_K)]
                for c4 in range(_C // _L):
                    prod = [ws[k] * rows_v[k * _CH + p, pl.ds(c4 * _L, _L)]
                            for k in range(_K)]
                    s01 = prod[0] + prod[1]
                    s23 = prod[2] + prod[3]
                    s45 = prod[4] + prod[5]
                    s67 = prod[6] + prod[7]
                    out_v[p, pl.ds(c4 * _L, _L)] = ((s01 + s23)
                                                    + (s45 + s67))

    load_and_fire(base_cid, bufs[0])

    def pair_body(h, carry):
        for b in range(2):
            j = 2 * h + b
            cid = base_cid + j
            buf = bufs[b]
            nxt = bufs[1 - b]

            @pl.when(j + 1 < cpw)
            def _():
                load_and_fire(cid + 1, nxt)

            wait_gathers(buf)

            @pl.when(j >= 2)
            def _():
                pltpu.make_async_copy(buf[3], out_hbm.at[cid - 2],
                                      buf[5]).wait()

            compute(buf)
            pltpu.async_copy(buf[3], out_hbm.at[cid], buf[5])
        return carry

    lax.fori_loop(0, cpw // 2, pair_body, 0)
    pltpu.make_async_copy(bufs[0][3], out_hbm.at[base_cid + cpw - 2],
                          bufs[0][5]).wait()
    pltpu.make_async_copy(bufs[1][3], out_hbm.at[base_cid + cpw - 1],
                          bufs[1][5]).wait()


def _make_call(tot):
    return pl.kernel(
        _body,
        out_type=jax.ShapeDtypeStruct((tot, _CH, _C), jnp.float32),
        mesh=plsc.VectorSubcoreMesh(core_axis_name="c", subcore_axis_name="s"),
        scratch_types=[
            pltpu.VMEM((_K, _CH), jnp.int32),
            pltpu.VMEM((_K, _CH), jnp.int32),
            pltpu.VMEM((_K, _CH), jnp.float32),
            pltpu.VMEM((_K, _CH), jnp.float32),
            pltpu.VMEM((_K * _CH, _C), jnp.float32),
            pltpu.VMEM((_K * _CH, _C), jnp.float32),
            pltpu.VMEM((_CH, _C), jnp.float32),
            pltpu.VMEM((_CH, _C), jnp.float32),
            pltpu.SemaphoreType.DMA,
            pltpu.SemaphoreType.DMA,
            pltpu.SemaphoreType.DMA,
            pltpu.SemaphoreType.DMA,
        ],
        compiler_params=pltpu.CompilerParams(use_tc_tiling_on_sc=False),
    )


def kernel(fragments, alphas, ptclds):
    n_img, k, h, w = fragments.shape
    c, p = ptclds.shape
    hw = h * w
    nblk = hw // _CH
    tot = n_img * nblk
    frag = fragments.astype(jnp.int32)
    alph = alphas
    table = ptclds.T
    out = _make_call(tot)(frag, alph, table)
    return (out.reshape(n_img, hw, c)
            .transpose(0, 2, 1)
            .reshape(n_img, c, h, w))


# speedup vs baseline: 1.7787x; 1.7787x over previous
import jax
import jax.numpy as jnp
from jax import lax
from jax.experimental import pallas as pl
from jax.experimental.pallas import tpu as pltpu
from jax.experimental.pallas import tpu_sc as plsc

_NC, _NS, _L = 2, 16, 16
_NW = _NC * _NS
_K = 8
_C = 64
_CH = 64
_NG = _K * _CH // 128


def _body(frag_hbm, alpha_hbm, table_hbm, out_hbm,
          idx0, idx1, alpha0, alpha1, rows0, rows1, out0, out1,
          gsem0, gsem1, osem0, osem1):
    tot = out_hbm.shape[0]
    wcb = frag_hbm.shape[3] // _CH
    nblk = frag_hbm.shape[2] * wcb
    cpw = tot // _NW
    wid = lax.axis_index("s") * _NC + lax.axis_index("c")
    base_cid = wid * cpw
    bufs = [(idx0, alpha0, rows0, out0, gsem0, osem0),
            (idx1, alpha1, rows1, out1, gsem1, osem1)]

    def load_and_fire(cid, buf):
        idx_v, alpha_v, rows_v = buf[0], buf[1], buf[2]
        n = cid // nblk
        blk = cid % nblk
        hh = blk // wcb
        w0 = (blk % wcb) * _CH
        pltpu.sync_copy(frag_hbm.at[n, :, hh, pl.ds(w0, _CH)], idx_v)
        pltpu.sync_copy(alpha_hbm.at[n, :, hh, pl.ds(w0, _CH)], alpha_v)
        for k in range(_K):
            pltpu.async_copy(table_hbm.at[idx_v.at[k]],
                             rows_v.at[pl.ds(k * _CH, _CH)], buf[4])

    def wait_gathers(buf):
        idx_v, rows_v = buf[0], buf[2]
        for k in range(_K):
            pltpu.make_async_copy(table_hbm.at[idx_v.at[k]],
                                  rows_v.at[pl.ds(k * _CH, _CH)],
                                  buf[4]).wait()

    def compute(buf):
        alpha_v, rows_v, out_v = buf[1], buf[2], buf[3]

        @plsc.parallel_loop(0, _CH // _L)
        def g_body(g):
            t = jnp.ones((_L,), jnp.float32)
            wk = []
            for k in range(_K):
                a = alpha_v[k, pl.ds(g * _L, _L)]
                wk.append(a * t)
                t = t * (1.0 - a)
            base = g * _L
            for j in range(_L):
                p = base + j
                ws = [wk[k][j] for k in range(_K)]
                for c4 in range(_C // _L):
                    prod = [ws[k] * rows_v[k * _CH + p, pl.ds(c4 * _L, _L)]
                            for k in range(_K)]
                    s01 = prod[0] + prod[1]
                    s23 = prod[2] + prod[3]
                    s45 = prod[4] + prod[5]
                    s67 = prod[6] + prod[7]
                    out_v[p, pl.ds(c4 * _L, _L)] = ((s01 + s23)
                                                    + (s45 + s67))

    load_and_fire(base_cid, bufs[0])

    def pair_body(h, carry):
        for b in range(2):
            j = 2 * h + b
            cid = base_cid + j
            buf = bufs[b]
            nxt = bufs[1 - b]

            @pl.when(j + 1 < cpw)
            def _():
                load_and_fire(cid + 1, nxt)

            wait_gathers(buf)

            @pl.when(j >= 2)
            def _():
                pltpu.make_async_copy(buf[3], out_hbm.at[cid - 2],
                                      buf[5]).wait()

            compute(buf)
            pltpu.async_copy(buf[3], out_hbm.at[cid], buf[5])
        return carry

    lax.fori_loop(0, cpw // 2, pair_body, 0)
    pltpu.make_async_copy(bufs[0][3], out_hbm.at[base_cid + cpw - 2],
                          bufs[0][5]).wait()
    pltpu.make_async_copy(bufs[1][3], out_hbm.at[base_cid + cpw - 1],
                          bufs[1][5]).wait()


def _make_call(tot):
    return pl.kernel(
        _body,
        out_type=jax.ShapeDtypeStruct((tot, _CH, _C), jnp.float32),
        mesh=plsc.VectorSubcoreMesh(core_axis_name="c", subcore_axis_name="s"),
        scratch_types=[
            pltpu.VMEM((_K, _CH), jnp.int32),
            pltpu.VMEM((_K, _CH), jnp.int32),
            pltpu.VMEM((_K, _CH), jnp.float32),
            pltpu.VMEM((_K, _CH), jnp.float32),
            pltpu.VMEM((_K * _CH, _C), jnp.float32),
            pltpu.VMEM((_K * _CH, _C), jnp.float32),
            pltpu.VMEM((_CH, _C), jnp.float32),
            pltpu.VMEM((_CH, _C), jnp.float32),
            pltpu.SemaphoreType.DMA,
            pltpu.SemaphoreType.DMA,
            pltpu.SemaphoreType.DMA,
            pltpu.SemaphoreType.DMA,
        ],
        compiler_params=pltpu.CompilerParams(use_tc_tiling_on_sc=False),
    )


def kernel(fragments, alphas, ptclds):
    n_img, k, h, w = fragments.shape
    c, p = ptclds.shape
    hw = h * w
    nblk = hw // _CH
    tot = n_img * nblk
    frag = fragments.astype(jnp.int32)
    alph = alphas
    table = ptclds.T
    out = _make_call(tot)(frag, alph, table)
    return (out.reshape(n_img, hw, c)
            .transpose(0, 2, 1)
            .reshape(n_img, c, h, w))
